# baseline (speedup 1.0000x reference)
import functools

import jax
import jax.numpy as jnp
from jax import lax
from jax.experimental import pallas as pl
from jax.experimental.pallas import tpu as pltpu
from jax.experimental.pallas import tpu_sc as plsc

_NC = 2
_NS = 16
_NW = _NC * _NS
_LANES = 16
_POS_CHUNK = 1024


def _sc_body(table_hbm, idx_hbm, out_hbm, my_rows, idx_v, sem_out,
             *, n, vocab):
    wid = lax.axis_index("s") * _NC + lax.axis_index("c")
    pair = wid // 2
    parity = wid % 2
    rpp = vocab // (_NW // 2)
    lo = pair * rpp

    pltpu.sync_copy(table_hbm.at[pl.ds(lo, rpp)], my_rows)

    def scan_chunk(c, cnt):
        pltpu.sync_copy(idx_hbm.at[pl.ds(c * _POS_CHUNK, _POS_CHUNK)], idx_v)

        def scan_vec(v, cnt_in):
            off = pl.multiple_of(v * _LANES, _LANES)
            lvec = idx_v[pl.ds(off, _LANES)] - lo
            hit = jnp.logical_and(lvec >= 0, lvec < rpp).astype(jnp.int32)
            pc = plsc.cumsum(hit)
            nhit = jnp.sum(hit, axis=0)
            ordinal = cnt_in + pc - hit
            take = jnp.logical_and(hit == 1, ordinal % 2 == parity)
            masked_l = jnp.where(take, lvec, -1)

            @pl.when(nhit > 0)
            def _lanes():
                for lane in range(_LANES):
                    l = masked_l[lane]

                    @pl.when(l >= 0)
                    def _push():
                        pltpu.async_copy(
                            my_rows.at[l],
                            out_hbm.at[c * _POS_CHUNK + v * _LANES + lane],
                            sem_out)

            return cnt_in + nhit

        return pl.loop(0, _POS_CHUNK // _LANES, init_carry=cnt)(scan_vec)

    total = pl.loop(0, n // _POS_CHUNK, init_carry=jnp.int32(0))(scan_chunk)
    mine = (total + (1 - parity)) // 2

    def drain(_i, carry):
        pltpu.make_async_copy(my_rows.at[0], out_hbm.at[0], sem_out).wait()
        return carry

    pl.loop(0, mine, init_carry=jnp.int32(0))(drain)


def kernel(prefix, table):
    bsz, seq = prefix.shape
    n = bsz * seq
    vocab, width = table.shape

    idx = prefix.reshape(n).astype(jnp.int32)
    mesh = plsc.VectorSubcoreMesh(core_axis_name="c", subcore_axis_name="s")
    body = functools.partial(_sc_body, n=n, vocab=vocab)
    k = pl.kernel(
        body,
        out_type=jax.ShapeDtypeStruct((n, width), table.dtype),
        mesh=mesh,
        compiler_params=pltpu.CompilerParams(needs_layout_passes=False),
        scratch_types=[
            pltpu.VMEM((vocab // (_NW // 2), width), table.dtype),
            pltpu.VMEM((_POS_CHUNK,), jnp.int32),
            pltpu.SemaphoreType.DMA,
        ],
    )
    out = k(table, idx)
    return out.reshape(bsz, seq, width)


# speedup vs baseline: 1.0766x; 1.0766x over previous
import functools

import jax
import jax.numpy as jnp
from jax import lax
from jax.experimental import pallas as pl
from jax.experimental.pallas import tpu as pltpu
from jax.experimental.pallas import tpu_sc as plsc

_NC = 2
_NS = 16
_NW = _NC * _NS
_POS_CHUNK = 1024


def _sc_body(table_hbm, idx_hbm, out_hbm, my_rows, idx_v, sem_out,
             *, n, vocab):
    wid = lax.axis_index("s") * _NC + lax.axis_index("c")
    rpt = vocab // _NW
    lo = wid * rpt

    pltpu.sync_copy(table_hbm.at[pl.ds(lo, rpt)], my_rows)

    pltpu.sync_copy(idx_hbm, idx_v)

    def scan_vec(v, cnt_in):
        off = pl.multiple_of(v * 16, 16)
        lvec = idx_v[pl.ds(off, 16)] - lo
        hit = jnp.logical_and(lvec >= 0, lvec < rpt)
        nhit = jnp.sum(hit.astype(jnp.int32), axis=0)
        masked_l = jnp.where(hit, lvec, -1)

        @pl.when(nhit > 0)
        def _lanes():
            for lane in range(16):
                l = masked_l[lane]

                @pl.when(l >= 0)
                def _push():
                    pltpu.async_copy(
                        my_rows.at[l], out_hbm.at[v * 16 + lane], sem_out)

        return cnt_in + nhit

    total = pl.loop(0, n // 16, init_carry=jnp.int32(0))(scan_vec)

    def drain(_i, carry):
        pltpu.make_async_copy(my_rows.at[0], out_hbm.at[0], sem_out).wait()
        return carry

    pl.loop(0, total, init_carry=jnp.int32(0))(drain)


def kernel(prefix, table):
    bsz, seq = prefix.shape
    n = bsz * seq
    vocab, width = table.shape

    idx = prefix.reshape(n).astype(jnp.int32)
    mesh = plsc.VectorSubcoreMesh(core_axis_name="c", subcore_axis_name="s")
    body = functools.partial(_sc_body, n=n, vocab=vocab)
    k = pl.kernel(
        body,
        out_type=jax.ShapeDtypeStruct((n, width), table.dtype),
        mesh=mesh,
        compiler_params=pltpu.CompilerParams(needs_layout_passes=False),
        scratch_types=[
            pltpu.VMEM((vocab // _NW, width), table.dtype),
            pltpu.VMEM((n,), jnp.int32),
            pltpu.SemaphoreType.DMA,
        ],
    )
    out = k(table, idx)
    return out.reshape(bsz, seq, width)


# speedup vs baseline: 1.1461x; 1.0645x over previous
import functools

import jax
import jax.numpy as jnp
from jax import lax
from jax.experimental import pallas as pl
from jax.experimental.pallas import tpu as pltpu
from jax.experimental.pallas import tpu_sc as plsc

_NC = 2
_NS = 16
_NW = _NC * _NS
_POS_CHUNK = 1024


def _sc_body(table_hbm, idx_hbm, out_hbm, my_rows, idx_v, sem_out,
             *, n, vocab):
    wid = lax.axis_index("s") * _NC + lax.axis_index("c")
    rpt = vocab // _NW
    lo = wid * rpt

    pltpu.sync_copy(table_hbm.at[pl.ds(lo, rpt)], my_rows)

    pltpu.sync_copy(idx_hbm, idx_v)

    def scan_vec(v, cnt_in):
        off = pl.multiple_of(v * 16, 16)
        lvec = idx_v[pl.ds(off, 16)] - lo
        hit = jnp.logical_and(lvec >= 0, lvec < rpt)
        hit_i = hit.astype(jnp.int32)
        half_sel = (lax.iota(jnp.int32, 16) < 8).astype(jnp.int32)
        nlow = jnp.sum(hit_i * half_sel, axis=0)
        nhit = jnp.sum(hit_i, axis=0)
        masked_l = jnp.where(hit, lvec, -1)

        @pl.when(nlow > 0)
        def _lanes_lo():
            for lane in range(8):
                l = masked_l[lane]

                @pl.when(l >= 0)
                def _push():
                    pltpu.async_copy(
                        my_rows.at[l], out_hbm.at[v * 16 + lane], sem_out)

        @pl.when(nhit - nlow > 0)
        def _lanes_hi():
            for lane in range(8, 16):
                l = masked_l[lane]

                @pl.when(l >= 0)
                def _push():
                    pltpu.async_copy(
                        my_rows.at[l], out_hbm.at[v * 16 + lane], sem_out)

        return cnt_in + nhit

    total = pl.loop(0, n // 16, init_carry=jnp.int32(0))(scan_vec)

    def drain(_i, carry):
        pltpu.make_async_copy(my_rows.at[0], out_hbm.at[0], sem_out).wait()
        return carry

    pl.loop(0, total, init_carry=jnp.int32(0))(drain)


def kernel(prefix, table):
    bsz, seq = prefix.shape
    n = bsz * seq
    vocab, width = table.shape

    idx = prefix.reshape(n).astype(jnp.int32)
    mesh = plsc.VectorSubcoreMesh(core_axis_name="c", subcore_axis_name="s")
    body = functools.partial(_sc_body, n=n, vocab=vocab)
    k = pl.kernel(
        body,
        out_type=jax.ShapeDtypeStruct((n, width), table.dtype),
        mesh=mesh,
        compiler_params=pltpu.CompilerParams(needs_layout_passes=False),
        scratch_types=[
            pltpu.VMEM((vocab // _NW, width), table.dtype),
            pltpu.VMEM((n,), jnp.int32),
            pltpu.SemaphoreType.DMA,
        ],
    )
    out = k(table, idx)
    return out.reshape(bsz, seq, width)


# speedup vs baseline: 1.1775x; 1.0274x over previous
import functools

import jax
import jax.numpy as jnp
from jax import lax
from jax.experimental import pallas as pl
from jax.experimental.pallas import tpu as pltpu
from jax.experimental.pallas import tpu_sc as plsc

_NC = 2
_NS = 16
_NW = _NC * _NS
_POS_CHUNK = 1024


def _sc_body(table_hbm, idx_hbm, out_hbm, my_rows, idx_v, sem_out,
             *, n, vocab):
    wid = lax.axis_index("s") * _NC + lax.axis_index("c")
    rpt = vocab // _NW
    lo = wid * rpt

    pltpu.sync_copy(table_hbm.at[pl.ds(lo, rpt)], my_rows)

    pltpu.sync_copy(idx_hbm, idx_v)

    def scan_vec(v, cnt_in):
        off = pl.multiple_of(v * 16, 16)
        lvec = idx_v[pl.ds(off, 16)] - lo
        hit = jnp.logical_and(lvec >= 0, lvec < rpt)
        hit_i = hit.astype(jnp.int32)
        quarter = lax.iota(jnp.int32, 16) // 4
        nq = [jnp.sum(hit_i * (quarter == q).astype(jnp.int32), axis=0)
              for q in range(4)]
        nhit = jnp.sum(hit_i, axis=0)
        masked_l = jnp.where(hit, lvec, -1)

        @pl.when(nhit > 0)
        def _lanes():
            for q in range(4):
                @pl.when(nq[q] > 0)
                def _quarter(q=q):
                    for lane in range(q * 4, q * 4 + 4):
                        l = masked_l[lane]

                        @pl.when(l >= 0)
                        def _push(l=l, lane=lane):
                            pltpu.async_copy(
                                my_rows.at[l], out_hbm.at[v * 16 + lane],
                                sem_out)

        return cnt_in + nhit

    total = pl.loop(0, n // 16, init_carry=jnp.int32(0))(scan_vec)

    def drain(_i, carry):
        pltpu.make_async_copy(my_rows.at[0], out_hbm.at[0], sem_out).wait()
        return carry

    pl.loop(0, total, init_carry=jnp.int32(0))(drain)


def kernel(prefix, table):
    bsz, seq = prefix.shape
    n = bsz * seq
    vocab, width = table.shape

    idx = prefix.reshape(n).astype(jnp.int32)
    mesh = plsc.VectorSubcoreMesh(core_axis_name="c", subcore_axis_name="s")
    body = functools.partial(_sc_body, n=n, vocab=vocab)
    k = pl.kernel(
        body,
        out_type=jax.ShapeDtypeStruct((n, width), table.dtype),
        mesh=mesh,
        compiler_params=pltpu.CompilerParams(needs_layout_passes=False),
        scratch_types=[
            pltpu.VMEM((vocab // _NW, width), table.dtype),
            pltpu.VMEM((n,), jnp.int32),
            pltpu.SemaphoreType.DMA,
        ],
    )
    out = k(table, idx)
    return out.reshape(bsz, seq, width)


# speedup vs baseline: 1.1855x; 1.0067x over previous
import functools

import jax
import jax.numpy as jnp
from jax import lax
from jax.experimental import pallas as pl
from jax.experimental.pallas import tpu as pltpu
from jax.experimental.pallas import tpu_sc as plsc

_NC = 2
_NS = 16
_NW = _NC * _NS
_POS_CHUNK = 1024


def _sc_body(table_hbm, idx_hbm, out_hbm, my_rows, idx_v, sem_out,
             *, n, vocab):
    wid = lax.axis_index("s") * _NC + lax.axis_index("c")
    parity = wid % 2
    rpt = vocab // (_NW // 2)
    lo = (wid // 2) * rpt

    pltpu.sync_copy(table_hbm.at[pl.ds(lo, rpt)], my_rows)

    pltpu.sync_copy(idx_hbm, idx_v)

    def scan_vec(v, cnt_in):
        off = pl.multiple_of(v * 16, 16)
        lvec = idx_v[pl.ds(off, 16)] - lo
        hit = jnp.logical_and(lvec >= 0, lvec < rpt)
        hit_i = hit.astype(jnp.int32)
        nhit = jnp.sum(hit_i, axis=0)
        ordinal = cnt_in + plsc.cumsum(hit_i) - hit_i
        take = jnp.logical_and(hit, ordinal % 2 == parity)
        take_i = take.astype(jnp.int32)
        quarter = lax.iota(jnp.int32, 16) // 4
        nq = [jnp.sum(take_i * (quarter == q).astype(jnp.int32), axis=0)
              for q in range(4)]
        masked_l = jnp.where(take, lvec, -1)

        @pl.when(jnp.sum(take_i, axis=0) > 0)
        def _lanes():
            for q in range(4):
                @pl.when(nq[q] > 0)
                def _quarter(q=q):
                    for lane in range(q * 4, q * 4 + 4):
                        l = masked_l[lane]

                        @pl.when(l >= 0)
                        def _push(l=l, lane=lane):
                            pltpu.async_copy(
                                my_rows.at[l], out_hbm.at[v * 16 + lane],
                                sem_out)

        return cnt_in + nhit

    total = pl.loop(0, n // 16, init_carry=jnp.int32(0))(scan_vec)
    total = (total + (1 - parity)) // 2

    def drain(_i, carry):
        pltpu.make_async_copy(my_rows.at[0], out_hbm.at[0], sem_out).wait()
        return carry

    pl.loop(0, total, init_carry=jnp.int32(0))(drain)


def kernel(prefix, table):
    bsz, seq = prefix.shape
    n = bsz * seq
    vocab, width = table.shape

    idx = prefix.reshape(n).astype(jnp.int32)
    mesh = plsc.VectorSubcoreMesh(core_axis_name="c", subcore_axis_name="s")
    body = functools.partial(_sc_body, n=n, vocab=vocab)
    k = pl.kernel(
        body,
        out_type=jax.ShapeDtypeStruct((n, width), table.dtype),
        mesh=mesh,
        compiler_params=pltpu.CompilerParams(needs_layout_passes=False),
        scratch_types=[
            pltpu.VMEM((vocab // (_NW // 2), width), table.dtype),
            pltpu.VMEM((n,), jnp.int32),
            pltpu.SemaphoreType.DMA,
        ],
    )
    out = k(table, idx)
    return out.reshape(bsz, seq, width)
